# Initial kernel scaffold; baseline (speedup 1.0000x reference)
import functools

import numpy as np
import jax
import jax.numpy as jnp
from jax import lax
from jax.experimental import pallas as pl
from jax.experimental.pallas import tpu as pltpu
from jax.experimental.pallas import tpu_sc as plsc

NC = 2
NS = 16
NW = NC * NS
LANES = 16
CHUNK = 1024


def kernel(x, input_embedding_table):
    B, H = x.shape
    V, D = input_embedding_table.shape
    n = B * H
    assert n % (NW * CHUNK) == 0 and D == 2 * LANES
    per_w = n // NW
    n_chunks = per_w // CHUNK
    scale = float(np.sqrt(np.float32(D)))

    idx = x.reshape(n)
    mesh = plsc.VectorSubcoreMesh(core_axis_name="c", subcore_axis_name="s")

    @functools.partial(
        pl.kernel,
        mesh=mesh,
        out_type=jax.ShapeDtypeStruct((n, D), jnp.float32),
        scratch_types=[
            pltpu.VMEM((CHUNK,), jnp.int32),
            pltpu.VMEM((CHUNK, D), jnp.float32),
            pltpu.SemaphoreType.DMA,
        ],
    )
    def sc_gather(table_hbm, idx_hbm, out_hbm, idx_v, rows_v, sem):
        wid = lax.axis_index("s") * NC + lax.axis_index("c")
        base = wid * per_w

        @pl.loop(0, n_chunks)
        def _(ci):
            off = base + ci * CHUNK
            pltpu.sync_copy(idx_hbm.at[pl.ds(off, CHUNK)], idx_v)
            pltpu.async_copy(table_hbm.at[idx_v], rows_v, sem).wait()

            @pl.loop(0, CHUNK)
            def _(r):
                rows_v[r, pl.ds(0, LANES)] = rows_v[r, pl.ds(0, LANES)] * scale
                rows_v[r, pl.ds(LANES, LANES)] = (
                    rows_v[r, pl.ds(LANES, LANES)] * scale
                )

            pltpu.sync_copy(rows_v, out_hbm.at[pl.ds(off, CHUNK)])

    out = sc_gather(input_embedding_table, idx)
    return out.reshape(B, H, D)


# speedup vs baseline: 1.2246x; 1.2246x over previous
import functools

import numpy as np
import jax
import jax.numpy as jnp
from jax import lax
from jax.experimental import pallas as pl
from jax.experimental.pallas import tpu as pltpu
from jax.experimental.pallas import tpu_sc as plsc

NC = 2
NS = 16
NW = NC * NS
LANES = 16
CHUNK = 1024


def kernel(x, input_embedding_table):
    B, H = x.shape
    V, D = input_embedding_table.shape
    n = B * H
    assert n % (NW * CHUNK) == 0 and D == 2 * LANES
    per_w = n // NW
    n_chunks = per_w // CHUNK
    scale = float(np.sqrt(np.float32(D)))

    idx = x.reshape(n)
    mesh = plsc.VectorSubcoreMesh(core_axis_name="c", subcore_axis_name="s")

    @functools.partial(
        pl.kernel,
        mesh=mesh,
        compiler_params=pltpu.CompilerParams(use_tc_tiling_on_sc=False),
        out_type=jax.ShapeDtypeStruct((n, D), jnp.float32),
        scratch_types=[
            pltpu.VMEM((CHUNK,), jnp.int32),
            pltpu.VMEM((CHUNK, D), jnp.float32),
            pltpu.SemaphoreType.DMA,
        ],
    )
    def sc_gather(table_hbm, idx_hbm, out_hbm, idx_v, rows_v, sem):
        wid = lax.axis_index("s") * NC + lax.axis_index("c")
        base = wid * per_w

        @pl.loop(0, n_chunks)
        def _(ci):
            off = base + ci * CHUNK
            pltpu.sync_copy(idx_hbm.at[pl.ds(off, CHUNK)], idx_v)
            pltpu.async_copy(table_hbm.at[idx_v], rows_v, sem).wait()

            @pl.loop(0, CHUNK)
            def _(r):
                rows_v[r, pl.ds(0, LANES)] = rows_v[r, pl.ds(0, LANES)] * scale
                rows_v[r, pl.ds(LANES, LANES)] = (
                    rows_v[r, pl.ds(LANES, LANES)] * scale
                )

            pltpu.sync_copy(rows_v, out_hbm.at[pl.ds(off, CHUNK)])

    out = sc_gather(input_embedding_table, idx)
    return out.reshape(B, H, D)


# speedup vs baseline: 1.3172x; 1.0756x over previous
import functools

import numpy as np
import jax
import jax.numpy as jnp
from jax import lax
from jax.experimental import pallas as pl
from jax.experimental.pallas import tpu as pltpu
from jax.experimental.pallas import tpu_sc as plsc

NC = 2
NS = 16
NW = NC * NS
LANES = 16
CHUNK = 1024
NBUF = 3


def kernel(x, input_embedding_table):
    B, H = x.shape
    V, D = input_embedding_table.shape
    n = B * H
    assert n % (NW * CHUNK) == 0 and D == 2 * LANES
    per_w = n // NW
    n_chunks = per_w // CHUNK
    scale = float(np.sqrt(np.float32(D)))

    idx = x.reshape(n)
    mesh = plsc.VectorSubcoreMesh(core_axis_name="c", subcore_axis_name="s")

    @functools.partial(
        pl.kernel,
        mesh=mesh,
        compiler_params=pltpu.CompilerParams(use_tc_tiling_on_sc=False),
        out_type=jax.ShapeDtypeStruct((n, D), jnp.float32),
        scratch_types=[
            pltpu.VMEM((per_w,), jnp.int32),
            pltpu.VMEM((NBUF, CHUNK, D), jnp.float32),
            pltpu.SemaphoreType.DMA((NBUF,)),
            pltpu.SemaphoreType.DMA((NBUF,)),
        ],
    )
    def sc_gather(table_hbm, idx_hbm, out_hbm, idx_v, rows_v, gsem, osem):
        wid = lax.axis_index("s") * NC + lax.axis_index("c")
        base = wid * per_w
        pltpu.sync_copy(idx_hbm.at[pl.ds(base, per_w)], idx_v)

        def start_gather(ci):
            b = ci % NBUF
            return pltpu.async_copy(
                table_hbm.at[idx_v.at[pl.ds(ci * CHUNK, CHUNK)]],
                rows_v.at[b],
                gsem.at[b],
            )

        def start_out(ci):
            b = ci % NBUF
            return pltpu.async_copy(
                rows_v.at[b],
                out_hbm.at[pl.ds(base + ci * CHUNK, CHUNK)],
                osem.at[b],
            )

        gathers = {0: start_gather(0), 1: start_gather(1)}
        outs = {}
        for ci in range(n_chunks):
            nxt = ci + 2
            if nxt < n_chunks:
                if nxt - NBUF in outs:
                    outs[nxt - NBUF].wait()
                gathers[nxt] = start_gather(nxt)
            gathers[ci].wait()
            b = ci % NBUF
            rb = rows_v.at[b]

            @plsc.parallel_loop(0, CHUNK, step=1, unroll=8)
            def _(r):
                rb[r, pl.ds(0, LANES)] = rb[r, pl.ds(0, LANES)] * scale
                rb[r, pl.ds(LANES, LANES)] = rb[r, pl.ds(LANES, LANES)] * scale

            outs[ci] = start_out(ci)
        for ci in range(max(0, n_chunks - NBUF), n_chunks):
            if ci in outs:
                outs[ci].wait()

    out = sc_gather(input_embedding_table, idx)
    return out.reshape(B, H, D)


# speedup vs baseline: 1.3328x; 1.0118x over previous
import dataclasses
import functools

import numpy as np
import jax
import jax.numpy as jnp
from jax import lax
from jax.experimental import pallas as pl
from jax.experimental.pallas import tpu as pltpu
from jax.experimental.pallas import tpu_sc as plsc

NC = 2
NS = 16
NW = NC * NS
LANES = 16
CHUNK = 1024
NBUF = 3


def kernel(x, input_embedding_table):
    B, H = x.shape
    V, D = input_embedding_table.shape
    n = B * H
    assert n % (NW * CHUNK) == 0 and D == 2 * LANES
    per_w = n // NW
    n_chunks = per_w // CHUNK
    scale = float(np.sqrt(np.float32(D)))

    tcol_full = V // 128
    tail_w = V - tcol_full * 128
    main = (tcol_full // NW) & ~1
    extra = tcol_full - main * NW

    table_t = input_embedding_table.T
    idx = x.reshape(n)
    mesh = plsc.VectorSubcoreMesh(core_axis_name="c", subcore_axis_name="s")

    @functools.partial(
        pl.kernel,
        mesh=mesh,
        compiler_params=dataclasses.replace(
            pltpu.CompilerParams(use_tc_tiling_on_sc=True),
            needs_layout_passes=False,
        ),
        out_type=jax.ShapeDtypeStruct((V * D,), jnp.float32),
        scratch_types=[
            pltpu.VMEM((4, 8, 128), jnp.float32),
            pltpu.VMEM((4, 8, 128), jnp.float32),
            pltpu.VMEM((128 * 32,), jnp.float32),
            pltpu.VMEM((128 * 32,), jnp.float32),
            pltpu.VMEM((4, 8, tail_w), jnp.float32),
            pltpu.VMEM((tail_w * 32,), jnp.float32),
            pltpu.SemaphoreType.DMA((2,)),
            pltpu.SemaphoreType.DMA((2,)),
        ],
    )
    def sc_transpose(
        tbl_hbm, out_hbm, in0_v, in1_v, ob0_v, ob1_v, tin_v, tout_v, isem, osem
    ):
        in_bufs = (in0_v, in1_v)
        out_bufs = (ob0_v, ob1_v)
        wid = lax.axis_index("s") * NC + lax.axis_index("c")
        iota = lax.iota(jnp.int32, LANES)
        dt_lo = iota // 8
        s_ix = iota % 8

        def col_in_start(c, b):
            for dt in range(4):
                pltpu.async_copy(
                    tbl_hbm.at[pl.ds(dt * 8, 8), pl.ds(c * 128, 128)],
                    in_bufs[b].at[dt],
                    isem.at[b],
                )

        def col_in_wait(b):
            for dt in range(4):
                pltpu.make_async_copy(
                    tbl_hbm.at[pl.ds(dt * 8, 8), pl.ds(0, 128)],
                    in_bufs[b].at[dt],
                    isem.at[b],
                ).wait()

        def out_wait(b, bufref):
            pltpu.make_async_copy(
                bufref,
                out_hbm.at[pl.ds(0, bufref.shape[0])],
                osem.at[b],
            ).wait()

        def transpose_into(inref, outref, width):
            @plsc.parallel_loop(0, width, unroll=4)
            def _(l):
                lv = jnp.full((LANES,), 0, jnp.int32) + l
                g0 = plsc.load_gather(inref, [dt_lo, s_ix, lv])
                g1 = plsc.load_gather(inref, [dt_lo + 2, s_ix, lv])
                outref[pl.ds(l * 32, LANES)] = g0 * scale
                outref[pl.ds(l * 32 + LANES, LANES)] = g1 * scale

        col_in_start(wid, 0)
        col_in_start(wid + NW, 1)

        @pl.loop(0, main, step=2)
        def _(j):
            for b in (0, 1):
                jj = j + b
                c = wid + jj * NW
                col_in_wait(b)

                @pl.when(jj >= 2)
                def _():
                    out_wait(b, out_bufs[b])

                transpose_into(in_bufs[b], out_bufs[b], 128)

                @pl.when(jj + 2 < main)
                def _():
                    col_in_start(wid + (jj + 2) * NW, b)

                pltpu.async_copy(
                    out_bufs[b],
                    out_hbm.at[pl.ds(c * 4096, 4096)],
                    osem.at[b],
                )

        out_wait(0, ob0_v)
        out_wait(1, ob1_v)

        @pl.when(wid < extra)
        def _():
            c = main * NW + wid
            for dt in range(4):
                pltpu.sync_copy(
                    tbl_hbm.at[pl.ds(dt * 8, 8), pl.ds(c * 128, 128)],
                    in0_v.at[dt],
                )
            transpose_into(in0_v, ob0_v, 128)
            pltpu.sync_copy(ob0_v, out_hbm.at[pl.ds(c * 4096, 4096)])

        @pl.when(wid == NW - 1)
        def _():
            for dt in range(4):
                pltpu.sync_copy(
                    tbl_hbm.at[pl.ds(dt * 8, 8), pl.ds(tcol_full * 128, tail_w)],
                    tin_v.at[dt],
                )
            transpose_into(tin_v, tout_v, tail_w)
            pltpu.sync_copy(
                tout_v, out_hbm.at[pl.ds(tcol_full * 4096, tail_w * 32)]
            )

    tbl_lin = sc_transpose(table_t)
    tbl2d = tbl_lin.reshape(V, D)

    @functools.partial(
        pl.kernel,
        mesh=mesh,
        compiler_params=pltpu.CompilerParams(use_tc_tiling_on_sc=False),
        out_type=jax.ShapeDtypeStruct((n, D), jnp.float32),
        scratch_types=[
            pltpu.VMEM((per_w,), jnp.int32),
            pltpu.VMEM((NBUF, CHUNK, D), jnp.float32),
            pltpu.SemaphoreType.DMA((NBUF,)),
            pltpu.SemaphoreType.DMA((NBUF,)),
        ],
    )
    def sc_gather(table_hbm, idx_hbm, out_hbm, idx_v, rows_v, gsem, osem):
        wid = lax.axis_index("s") * NC + lax.axis_index("c")
        base = wid * per_w
        pltpu.sync_copy(idx_hbm.at[pl.ds(base, per_w)], idx_v)

        def start_gather(ci):
            b = ci % NBUF
            return pltpu.async_copy(
                table_hbm.at[idx_v.at[pl.ds(ci * CHUNK, CHUNK)]],
                rows_v.at[b],
                gsem.at[b],
            )

        def start_out(ci):
            b = ci % NBUF
            return pltpu.async_copy(
                rows_v.at[b],
                out_hbm.at[pl.ds(base + ci * CHUNK, CHUNK)],
                osem.at[b],
            )

        gathers = {0: start_gather(0), 1: start_gather(1)}
        outs = {}
        for ci in range(n_chunks):
            nxt = ci + 2
            if nxt < n_chunks:
                if nxt - NBUF in outs:
                    outs[nxt - NBUF].wait()
                gathers[nxt] = start_gather(nxt)
            gathers[ci].wait()
            outs[ci] = start_out(ci)
        for ci in range(max(0, n_chunks - NBUF), n_chunks):
            if ci in outs:
                outs[ci].wait()

    out = sc_gather(tbl2d, idx)
    return out.reshape(B, H, D)
